# speedup vs baseline: 1.0017x; 1.0017x over previous
import functools
import math

import jax
import jax.numpy as jnp
from jax import lax
from jax.experimental import pallas as pl
from jax.experimental.pallas import tpu as pltpu
from jax.experimental.pallas import tpu_sc as plsc

D_MODEL = 512
SCALE = math.sqrt(D_MODEL)

_info = plsc.get_sparse_core_info()
NC = _info.num_cores
NS = _info.num_subcores
NW = NC * NS

B = 4 * 2048
B_PER_W = B // NW
CHUNK = 64
NCHUNK = B_PER_W // CHUNK
VECS_PER_ROW = D_MODEL // 16

_mesh = plsc.VectorSubcoreMesh(core_axis_name="c", subcore_axis_name="s")


@functools.partial(
    pl.kernel,
    mesh=_mesh,
    out_type=jax.ShapeDtypeStruct((B, D_MODEL), jnp.float32),
    scratch_types=[
        pltpu.VMEM((B_PER_W,), jnp.int32),
        pltpu.VMEM((CHUNK, D_MODEL), jnp.float32),
        pltpu.SemaphoreType.DMA,
    ],
)
def _emb_lookup(x_hbm, table_hbm, out_hbm, idx_v, rows_v, gsem):
    wid = lax.axis_index("s") * NC + lax.axis_index("c")
    base = wid * B_PER_W
    pltpu.sync_copy(x_hbm.at[pl.ds(base, B_PER_W)], idx_v)
    for c in range(NCHUNK):
        pltpu.async_copy(
            table_hbm.at[idx_v.at[pl.ds(c * CHUNK, CHUNK)]], rows_v, gsem
        ).wait()

        @plsc.parallel_loop(0, CHUNK, unroll=2)
        def _scale_row(i):
            for j in range(VECS_PER_ROW):
                rows_v[i, pl.ds(j * 16, 16)] = rows_v[i, pl.ds(j * 16, 16)] * SCALE

        pltpu.sync_copy(rows_v, out_hbm.at[pl.ds(base + c * CHUNK, CHUNK)])


@jax.jit
def kernel(x, table):
    xf = x.reshape(-1).astype(jnp.int32)
    out = _emb_lookup(xf, table)
    return out.reshape(x.shape + (D_MODEL,))


# speedup vs baseline: 1.1614x; 1.1595x over previous
import functools
import math

import jax
import jax.numpy as jnp
from jax import lax
from jax.experimental import pallas as pl
from jax.experimental.pallas import tpu as pltpu
from jax.experimental.pallas import tpu_sc as plsc

D_MODEL = 512
SCALE = math.sqrt(D_MODEL)

_info = plsc.get_sparse_core_info()
NC = _info.num_cores
NS = _info.num_subcores
NW = NC * NS

B = 4 * 2048
B_PER_W = B // NW
CHUNK = 64
NCHUNK = B_PER_W // CHUNK
VECS_PER_ROW = D_MODEL // 16

_mesh = plsc.VectorSubcoreMesh(core_axis_name="c", subcore_axis_name="s")


@functools.partial(
    pl.kernel,
    mesh=_mesh,
    out_type=jax.ShapeDtypeStruct((B, D_MODEL), jnp.float32),
    scratch_types=[
        pltpu.VMEM((B_PER_W,), jnp.int32),
        pltpu.VMEM((2, CHUNK, D_MODEL), jnp.float32),
        pltpu.SemaphoreType.DMA,
        pltpu.SemaphoreType.DMA,
        pltpu.SemaphoreType.DMA,
        pltpu.SemaphoreType.DMA,
    ],
)
def _emb_lookup(x_hbm, table_hbm, out_hbm, idx_v, rows_v, g0, g1, s0, s1):
    wid = lax.axis_index("s") * NC + lax.axis_index("c")
    base = wid * B_PER_W
    gsem = [g0, g1]
    ssem = [s0, s1]
    pltpu.sync_copy(x_hbm.at[pl.ds(base, B_PER_W)], idx_v)

    def gather(c):
        return pltpu.async_copy(
            table_hbm.at[idx_v.at[pl.ds(c * CHUNK, CHUNK)]],
            rows_v.at[c % 2],
            gsem[c % 2],
        )

    def store(c):
        return pltpu.async_copy(
            rows_v.at[c % 2],
            out_hbm.at[pl.ds(base + c * CHUNK, CHUNK)],
            ssem[c % 2],
        )

    gathers = [gather(0)] + [None] * (NCHUNK - 1)
    stores = [None] * NCHUNK
    for c in range(NCHUNK):
        b = c % 2
        gathers[c].wait()
        if c + 1 < NCHUNK:
            if c >= 1:
                stores[c - 1].wait()
            gathers[c + 1] = gather(c + 1)

        @plsc.parallel_loop(0, CHUNK, unroll=2)
        def _scale_row(i):
            buf = rows_v.at[b]
            for j in range(VECS_PER_ROW):
                buf[i, pl.ds(j * 16, 16)] = buf[i, pl.ds(j * 16, 16)] * SCALE

        stores[c] = store(c)
    stores[NCHUNK - 2].wait()
    stores[NCHUNK - 1].wait()


@jax.jit
def kernel(x, table):
    xf = x.reshape(-1).astype(jnp.int32)
    out = _emb_lookup(xf, table)
    return out.reshape(x.shape + (D_MODEL,))
